# baseline (speedup 1.0000x reference)
import functools

import jax
import jax.numpy as jnp
from jax import lax
from jax.experimental import pallas as pl
from jax.experimental.pallas import tpu as pltpu
from jax.experimental.pallas import tpu_sc as plsc

_NC = 2
_NS = 16
_NW = _NC * _NS
_L = 16


@functools.lru_cache(maxsize=None)
def _build(B, NUMF, CATF, V, D):
    NTOK = 1 + NUMF + CATF
    BPW = B // _NW
    SUB = 16
    NSUB = BPW // SUB
    ND = D // _L

    mesh = plsc.VectorSubcoreMesh(core_axis_name="c", subcore_axis_name="s")

    @functools.partial(
        pl.kernel,
        out_type=jax.ShapeDtypeStruct((B, NTOK // 2, 2 * D), jnp.float32),
        mesh=mesh,
        scratch_types=[
            pltpu.VMEM((NUMF, D), jnp.float32),
            pltpu.VMEM((NUMF, D), jnp.float32),
            pltpu.VMEM((D,), jnp.float32),
            pltpu.VMEM((NUMF, BPW), jnp.float32),
            pltpu.VMEM((CATF, BPW), jnp.int32),
            pltpu.VMEM((CATF, BPW), jnp.int32),
            pltpu.VMEM((2 * _L, 8, D), jnp.float32),
            pltpu.VMEM((1, SUB, NTOK // 2, 2 * D), jnp.float32),
            pltpu.SemaphoreType.DMA,
            pltpu.SemaphoreType.DMA,
        ],
    )
    def sc_kernel(tab_hbm, catT_hbm, numT_hbm, w_hbm, bias_hbm, cls_hbm,
                  out_hbm, wv, bv, clsv, numv, g8m, r8m, ring, stg,
                  gsem, wsem):
        wid = lax.axis_index("s") * _NC + lax.axis_index("c")
        base = wid * BPW

        pltpu.sync_copy(catT_hbm.at[:, pl.ds(base, BPW)], g8m)
        for c in range(CATF):
            off = c * V
            for p in range(BPW // _L):
                sl = pl.ds(p * _L, _L)
                idx = g8m[c, sl] + off
                g8m[c, sl] = lax.bitwise_and(idx, ~7)
                r8m[c, sl] = lax.bitwise_and(idx, 7)

        pltpu.sync_copy(w_hbm, wv)
        pltpu.sync_copy(bias_hbm, bv)
        pltpu.sync_copy(cls_hbm, clsv)
        pltpu.sync_copy(numT_hbm.at[:, pl.ds(base, BPW)], numv)

        def issue_field(c, s):
            g8v = g8m[c, pl.ds(s * SUB, SUB)]
            sb = lax.rem(c, 2) * _L
            for j in range(_L):
                g8 = pl.multiple_of(g8v[j], 8)
                pltpu.async_copy(
                    tab_hbm.at[pl.ds(g8, 8), :], ring.at[sb + j], gsem)

        def chunk_body(s, carry):
            dbuf = 0
            issue_field(0, s)

            @pl.when(s > 0)
            def _wdrain():
                pltpu.make_async_copy(
                    stg.at[0], out_hbm.at[pl.ds(base, SUB)], wsem).wait()

            for j in range(SUB):
                for dd in range(ND):
                    sl = pl.ds(dd * _L, _L)
                    stg[dbuf, j, 0, sl] = clsv[sl]
            for i in range(NUMF):
                t = 1 + i
                pr = t // 2
                hoff = (t % 2) * D
                vec = numv[i, pl.ds(s * SUB, SUB)]
                for j in range(SUB):
                    x = vec[j]
                    for dd in range(ND):
                        sl = pl.ds(dd * _L, _L)
                        stg[dbuf, j, pr, pl.ds(hoff + dd * _L, _L)] = (
                            wv[i, sl] * x + bv[i, sl])

            def blk_body(c, c2):
                @pl.when((c > 0) & (c < CATF))
                def _issue():
                    issue_field(c, s)

                @pl.when(c > 0)
                def _extract():
                    cp = c - 1
                    pltpu.make_async_copy(
                        tab_hbm.at[pl.ds(0, _L * 8), :],
                        ring.at[pl.ds(0, _L)], gsem).wait()
                    r8v = r8m[cp, pl.ds(s * SUB, SUB)]
                    sb = lax.rem(cp, 2) * _L
                    pr = (1 + NUMF + cp) // 2
                    hoff = lax.rem(cp, 2) * D
                    for j in range(_L):
                        r = r8v[j]
                        for dd in range(ND):
                            sl = pl.ds(dd * _L, _L)
                            stg[dbuf, j, pr, pl.ds(hoff + dd * _L, _L)] = ring[
                                sb + j, r, sl]

                return c2

            lax.fori_loop(1, CATF + 1, blk_body, 0)

            pltpu.async_copy(
                stg.at[dbuf], out_hbm.at[pl.ds(base + s * SUB, SUB)], wsem)
            return carry

        lax.fori_loop(0, NSUB, chunk_body, 0)
        pltpu.make_async_copy(
            stg.at[0], out_hbm.at[pl.ds(base, SUB)], wsem).wait()

    return sc_kernel


def kernel(numerical, categorical, W_num, b_num, tables, cls_token):
    B, NUMF = numerical.shape
    CATF = categorical.shape[1]
    V, D = tables.shape[1], tables.shape[2]
    NTOK = 1 + NUMF + CATF
    tab_flat = tables.reshape(CATF * V, D)
    cat_t = categorical.T.astype(jnp.int32)
    num_t = numerical.T
    cls_vec = cls_token.reshape(D)
    fn = _build(B, NUMF, CATF, V, D)
    out_pair = fn(tab_flat, cat_t, num_t, W_num, b_num, cls_vec)
    return out_pair.reshape(B, NTOK, D)


# speedup vs baseline: 1.0241x; 1.0241x over previous
import functools

import jax
import jax.numpy as jnp
from jax import lax
from jax.experimental import pallas as pl
from jax.experimental.pallas import tpu as pltpu
from jax.experimental.pallas import tpu_sc as plsc

_NC = 2
_NS = 16
_NW = _NC * _NS
_L = 16


@functools.lru_cache(maxsize=None)
def _build(B, NUMF, CATF, V, D):
    NTOK = 1 + NUMF + CATF
    BPW = B // _NW
    SUB = 16
    NSUB = BPW // SUB
    ND = D // _L

    mesh = plsc.VectorSubcoreMesh(core_axis_name="c", subcore_axis_name="s")

    @functools.partial(
        pl.kernel,
        out_type=jax.ShapeDtypeStruct((B, NTOK // 2, 2 * D), jnp.float32),
        mesh=mesh,
        scratch_types=[
            pltpu.VMEM((NUMF, D), jnp.float32),
            pltpu.VMEM((NUMF, D), jnp.float32),
            pltpu.VMEM((D,), jnp.float32),
            pltpu.VMEM((NUMF, BPW), jnp.float32),
            pltpu.VMEM((CATF, BPW), jnp.int32),
            pltpu.VMEM((CATF, BPW), jnp.int32),
            pltpu.VMEM((2 * _L, 8, D), jnp.float32),
            pltpu.VMEM((1, SUB, NTOK // 2, 2 * D), jnp.float32),
            pltpu.SemaphoreType.DMA,
            pltpu.SemaphoreType.DMA,
        ],
    )
    def sc_kernel(tab_hbm, catT_hbm, numT_hbm, w_hbm, bias_hbm, cls_hbm,
                  out_hbm, wv, bv, clsv, numv, g8m, r8m, ring, stg,
                  gsem, wsem):
        wid = lax.axis_index("s") * _NC + lax.axis_index("c")
        base = wid * BPW

        pltpu.sync_copy(catT_hbm.at[:, pl.ds(base, BPW)], g8m)
        for c in range(CATF):
            off = c * V
            for p in range(BPW // _L):
                sl = pl.ds(p * _L, _L)
                idx = g8m[c, sl] + off
                g8m[c, sl] = lax.bitwise_and(idx, ~7)
                r8m[c, sl] = lax.bitwise_and(idx, 7)

        pltpu.sync_copy(w_hbm, wv)
        pltpu.sync_copy(bias_hbm, bv)
        pltpu.sync_copy(cls_hbm, clsv)
        pltpu.sync_copy(numT_hbm.at[:, pl.ds(base, BPW)], numv)

        def issue_field(c, s):
            g8v = g8m[c, pl.ds(s * SUB, SUB)]
            sb = lax.rem(c, 2) * _L
            for j in range(_L):
                g8 = pl.multiple_of(g8v[j], 8)
                pltpu.async_copy(
                    tab_hbm.at[pl.ds(g8, 8), :], ring.at[sb + j], gsem)

        def chunk_body(s, carry):
            dbuf = 0
            issue_field(0, s)

            @pl.when(s > 0)
            def _wdrain():
                pltpu.make_async_copy(
                    stg.at[0], out_hbm.at[pl.ds(base, SUB)], wsem).wait()

            for j in range(SUB):
                for dd in range(ND):
                    sl = pl.ds(dd * _L, _L)
                    stg[dbuf, j, 0, sl] = clsv[sl]
            for i in range(NUMF):
                t = 1 + i
                pr = t // 2
                hoff = (t % 2) * D
                vec = numv[i, pl.ds(s * SUB, SUB)]
                for j in range(SUB):
                    x = vec[j]
                    for dd in range(ND):
                        sl = pl.ds(dd * _L, _L)
                        stg[dbuf, j, pr, pl.ds(hoff + dd * _L, _L)] = (
                            wv[i, sl] * x + bv[i, sl])

            def blk_body(c, c2):
                @pl.when((c > 0) & (c < CATF))
                def _issue():
                    issue_field(c, s)

                @pl.when(c > 0)
                def _extract():
                    cp = c - 1
                    r8v = r8m[cp, pl.ds(s * SUB, SUB)]
                    sb = lax.rem(cp, 2) * _L
                    pr = (1 + NUMF + cp) // 2
                    hoff = lax.rem(cp, 2) * D
                    for j in range(_L):
                        pltpu.make_async_copy(
                            tab_hbm.at[pl.ds(0, 8), :], ring.at[0], gsem).wait()
                        r = r8v[j]
                        for dd in range(ND):
                            sl = pl.ds(dd * _L, _L)
                            stg[dbuf, j, pr, pl.ds(hoff + dd * _L, _L)] = ring[
                                sb + j, r, sl]

                return c2

            lax.fori_loop(1, CATF + 1, blk_body, 0)

            pltpu.async_copy(
                stg.at[dbuf], out_hbm.at[pl.ds(base + s * SUB, SUB)], wsem)
            return carry

        lax.fori_loop(0, NSUB, chunk_body, 0)
        pltpu.make_async_copy(
            stg.at[0], out_hbm.at[pl.ds(base, SUB)], wsem).wait()

    return sc_kernel


def kernel(numerical, categorical, W_num, b_num, tables, cls_token):
    B, NUMF = numerical.shape
    CATF = categorical.shape[1]
    V, D = tables.shape[1], tables.shape[2]
    NTOK = 1 + NUMF + CATF
    tab_flat = tables.reshape(CATF * V, D)
    cat_t = categorical.T.astype(jnp.int32)
    num_t = numerical.T
    cls_vec = cls_token.reshape(D)
    fn = _build(B, NUMF, CATF, V, D)
    out_pair = fn(tab_flat, cat_t, num_t, W_num, b_num, cls_vec)
    return out_pair.reshape(B, NTOK, D)


# speedup vs baseline: 1.0623x; 1.0373x over previous
import functools

import jax
import jax.numpy as jnp
from jax import lax
from jax.experimental import pallas as pl
from jax.experimental.pallas import tpu as pltpu
from jax.experimental.pallas import tpu_sc as plsc

_NC = 2
_NS = 16
_NW = _NC * _NS
_L = 16


@functools.lru_cache(maxsize=None)
def _build(B, NUMF, CATF, V, D):
    NTOK = 1 + NUMF + CATF
    BPW = B // _NW
    SUB = 16
    NSUB = BPW // SUB
    ND = D // _L

    mesh = plsc.VectorSubcoreMesh(core_axis_name="c", subcore_axis_name="s")

    @functools.partial(
        pl.kernel,
        out_type=jax.ShapeDtypeStruct((B, NTOK // 2, 2 * D), jnp.float32),
        mesh=mesh,
        scratch_types=[
            pltpu.VMEM((NUMF, D), jnp.float32),
            pltpu.VMEM((NUMF, D), jnp.float32),
            pltpu.VMEM((D,), jnp.float32),
            pltpu.VMEM((NUMF, BPW), jnp.float32),
            pltpu.VMEM((CATF, BPW), jnp.int32),
            pltpu.VMEM((CATF, BPW), jnp.int32),
            pltpu.VMEM((3 * _L, 8, D), jnp.float32),
            pltpu.VMEM((1, SUB, NTOK // 2, 2 * D), jnp.float32),
            pltpu.SemaphoreType.DMA,
            pltpu.SemaphoreType.DMA,
        ],
    )
    def sc_kernel(tab_hbm, catT_hbm, numT_hbm, w_hbm, bias_hbm, cls_hbm,
                  out_hbm, wv, bv, clsv, numv, g8m, r8m, ring, stg,
                  gsem, wsem):
        wid = lax.axis_index("s") * _NC + lax.axis_index("c")
        base = wid * BPW

        pltpu.sync_copy(catT_hbm.at[:, pl.ds(base, BPW)], g8m)
        for c in range(CATF):
            off = c * V
            for p in range(BPW // _L):
                sl = pl.ds(p * _L, _L)
                idx = g8m[c, sl] + off
                g8m[c, sl] = lax.bitwise_and(idx, ~7)
                r8m[c, sl] = lax.bitwise_and(idx, 7)

        pltpu.sync_copy(w_hbm, wv)
        pltpu.sync_copy(bias_hbm, bv)
        pltpu.sync_copy(cls_hbm, clsv)
        pltpu.sync_copy(numT_hbm.at[:, pl.ds(base, BPW)], numv)

        def issue_field(c, s):
            g8v = g8m[c, pl.ds(s * SUB, SUB)]
            sb = lax.rem(c, 3) * _L
            for j in range(_L):
                g8 = pl.multiple_of(g8v[j], 8)
                pltpu.async_copy(
                    tab_hbm.at[pl.ds(g8, 8), :], ring.at[sb + j], gsem)

        def chunk_body(s, carry):
            dbuf = 0
            issue_field(0, s)
            issue_field(1, s)

            @pl.when(s > 0)
            def _wdrain():
                pltpu.make_async_copy(
                    stg.at[0], out_hbm.at[pl.ds(base, SUB)], wsem).wait()

            for j in range(SUB):
                for dd in range(ND):
                    sl = pl.ds(dd * _L, _L)
                    stg[dbuf, j, 0, sl] = clsv[sl]
            for i in range(NUMF):
                t = 1 + i
                pr = t // 2
                hoff = (t % 2) * D
                vec = numv[i, pl.ds(s * SUB, SUB)]
                for j in range(SUB):
                    x = vec[j]
                    for dd in range(ND):
                        sl = pl.ds(dd * _L, _L)
                        stg[dbuf, j, pr, pl.ds(hoff + dd * _L, _L)] = (
                            wv[i, sl] * x + bv[i, sl])

            def blk_body(c, c2):
                @pl.when(c + 1 < CATF)
                def _issue():
                    issue_field(c + 1, s)

                @pl.when(c > 0)
                def _extract():
                    cp = c - 1
                    r8v = r8m[cp, pl.ds(s * SUB, SUB)]
                    sb = lax.rem(cp, 3) * _L
                    pr = (1 + NUMF + cp) // 2
                    hoff = lax.rem(cp, 2) * D
                    for j in range(_L):
                        pltpu.make_async_copy(
                            tab_hbm.at[pl.ds(0, 8), :], ring.at[0], gsem).wait()
                        r = r8v[j]
                        for dd in range(ND):
                            sl = pl.ds(dd * _L, _L)
                            stg[dbuf, j, pr, pl.ds(hoff + dd * _L, _L)] = ring[
                                sb + j, r, sl]

                return c2

            lax.fori_loop(1, CATF + 1, blk_body, 0)

            pltpu.async_copy(
                stg.at[dbuf], out_hbm.at[pl.ds(base + s * SUB, SUB)], wsem)
            return carry

        lax.fori_loop(0, NSUB, chunk_body, 0)
        pltpu.make_async_copy(
            stg.at[0], out_hbm.at[pl.ds(base, SUB)], wsem).wait()

    return sc_kernel


def kernel(numerical, categorical, W_num, b_num, tables, cls_token):
    B, NUMF = numerical.shape
    CATF = categorical.shape[1]
    V, D = tables.shape[1], tables.shape[2]
    NTOK = 1 + NUMF + CATF
    tab_flat = tables.reshape(CATF * V, D)
    cat_t = categorical.T.astype(jnp.int32)
    num_t = numerical.T
    cls_vec = cls_token.reshape(D)
    fn = _build(B, NUMF, CATF, V, D)
    out_pair = fn(tab_flat, cat_t, num_t, W_num, b_num, cls_vec)
    return out_pair.reshape(B, NTOK, D)


# speedup vs baseline: 1.0688x; 1.0061x over previous
import functools

import jax
import jax.numpy as jnp
from jax import lax
from jax.experimental import pallas as pl
from jax.experimental.pallas import tpu as pltpu
from jax.experimental.pallas import tpu_sc as plsc

_NC = 2
_NS = 16
_NW = _NC * _NS
_L = 16


@functools.lru_cache(maxsize=None)
def _build(B, NUMF, CATF, V, D):
    NTOK = 1 + NUMF + CATF
    BPW = B // _NW
    SUB = 16
    NSUB = BPW // SUB
    ND = D // _L

    mesh = plsc.VectorSubcoreMesh(core_axis_name="c", subcore_axis_name="s")

    @functools.partial(
        pl.kernel,
        out_type=jax.ShapeDtypeStruct((B, NTOK // 2, 2 * D), jnp.float32),
        mesh=mesh,
        scratch_types=[
            pltpu.VMEM((NUMF, D), jnp.float32),
            pltpu.VMEM((NUMF, D), jnp.float32),
            pltpu.VMEM((D,), jnp.float32),
            pltpu.VMEM((NUMF, BPW), jnp.float32),
            pltpu.VMEM((CATF, BPW), jnp.int32),
            pltpu.VMEM((CATF, BPW), jnp.int32),
            pltpu.VMEM((4 * _L, 8, D), jnp.float32),
            pltpu.VMEM((1, SUB, NTOK // 2, 2 * D), jnp.float32),
            pltpu.SemaphoreType.DMA,
            pltpu.SemaphoreType.DMA,
        ],
    )
    def sc_kernel(tab_hbm, catT_hbm, numT_hbm, w_hbm, bias_hbm, cls_hbm,
                  out_hbm, wv, bv, clsv, numv, g8m, r8m, ring, stg,
                  gsem, wsem):
        wid = lax.axis_index("s") * _NC + lax.axis_index("c")
        base = wid * BPW

        pltpu.sync_copy(catT_hbm.at[:, pl.ds(base, BPW)], g8m)
        for c in range(CATF):
            off = c * V
            for p in range(BPW // _L):
                sl = pl.ds(p * _L, _L)
                idx = g8m[c, sl] + off
                g8m[c, sl] = lax.bitwise_and(idx, ~7)
                r8m[c, sl] = lax.bitwise_and(idx, 7)

        pltpu.sync_copy(w_hbm, wv)
        pltpu.sync_copy(bias_hbm, bv)
        pltpu.sync_copy(cls_hbm, clsv)
        pltpu.sync_copy(numT_hbm.at[:, pl.ds(base, BPW)], numv)

        def issue_field(c, s):
            g8v = g8m[c, pl.ds(s * SUB, SUB)]
            sb = lax.rem(c, 4) * _L
            for j in range(_L):
                g8 = pl.multiple_of(g8v[j], 8)
                pltpu.async_copy(
                    tab_hbm.at[pl.ds(g8, 8), :], ring.at[sb + j], gsem)

        def chunk_body(s, carry):
            dbuf = 0
            issue_field(0, s)
            issue_field(1, s)
            issue_field(2, s)

            @pl.when(s > 0)
            def _wdrain():
                pltpu.make_async_copy(
                    stg.at[0], out_hbm.at[pl.ds(base, SUB)], wsem).wait()

            for j in range(SUB):
                for dd in range(ND):
                    sl = pl.ds(dd * _L, _L)
                    stg[dbuf, j, 0, sl] = clsv[sl]
            for i in range(NUMF):
                t = 1 + i
                pr = t // 2
                hoff = (t % 2) * D
                vec = numv[i, pl.ds(s * SUB, SUB)]
                for j in range(SUB):
                    x = vec[j]
                    for dd in range(ND):
                        sl = pl.ds(dd * _L, _L)
                        stg[dbuf, j, pr, pl.ds(hoff + dd * _L, _L)] = (
                            wv[i, sl] * x + bv[i, sl])

            def blk_body(c, c2):
                @pl.when(c + 2 < CATF)
                def _issue():
                    issue_field(c + 2, s)

                @pl.when(c > 0)
                def _extract():
                    cp = c - 1
                    r8v = r8m[cp, pl.ds(s * SUB, SUB)]
                    sb = lax.rem(cp, 4) * _L
                    pr = (1 + NUMF + cp) // 2
                    hoff = lax.rem(cp, 2) * D
                    for j in range(_L):
                        pltpu.make_async_copy(
                            tab_hbm.at[pl.ds(0, 8), :], ring.at[0], gsem).wait()
                        r = r8v[j]
                        for dd in range(ND):
                            sl = pl.ds(dd * _L, _L)
                            stg[dbuf, j, pr, pl.ds(hoff + dd * _L, _L)] = ring[
                                sb + j, r, sl]

                return c2

            lax.fori_loop(1, CATF + 1, blk_body, 0)

            pltpu.async_copy(
                stg.at[dbuf], out_hbm.at[pl.ds(base + s * SUB, SUB)], wsem)
            return carry

        lax.fori_loop(0, NSUB, chunk_body, 0)
        pltpu.make_async_copy(
            stg.at[0], out_hbm.at[pl.ds(base, SUB)], wsem).wait()

    return sc_kernel


def kernel(numerical, categorical, W_num, b_num, tables, cls_token):
    B, NUMF = numerical.shape
    CATF = categorical.shape[1]
    V, D = tables.shape[1], tables.shape[2]
    NTOK = 1 + NUMF + CATF
    tab_flat = tables.reshape(CATF * V, D)
    cat_t = categorical.T.astype(jnp.int32)
    num_t = numerical.T
    cls_vec = cls_token.reshape(D)
    fn = _build(B, NUMF, CATF, V, D)
    out_pair = fn(tab_flat, cat_t, num_t, W_num, b_num, cls_vec)
    return out_pair.reshape(B, NTOK, D)
